# Initial kernel scaffold; baseline (speedup 1.0000x reference)
import functools

import jax
import jax.numpy as jnp
from jax import lax
from jax.experimental import pallas as pl
from jax.experimental.pallas import tpu as pltpu
from jax.experimental.pallas import tpu_sc as plsc

NUSERS = 20000
NN = 50000
DD = 64
EE = 800000
NC, NS, LANES = 2, 16, 16
HALF = NN // 2
TRASH = HALF
ACC_ROWS = 16 * 1568
DEG_WORDS = 16 * 3128

CH = 128

_mesh = plsc.VectorSubcoreMesh(
    core_axis_name="c", subcore_axis_name="s", num_cores=NC, num_subcores=NS)


def _deg_body(ei_hbm, ones_hbm, z_hbm, deg_out, didx, didx_t, ones_v, zv,
              acc):
    c = lax.axis_index("c")
    s = lax.axis_index("s")
    pltpu.sync_copy(z_hbm, zv)
    pltpu.sync_copy(zv, acc.at[pl.ds(pl.multiple_of(s * 3128, 8), 3128), :])
    pltpu.sync_copy(ones_hbm, ones_v)
    plsc.subcore_barrier()

    per_tile = EE // (NC * NS)
    base = (c * NS + s) * per_tile
    nfull = per_tile // CH
    tail = per_tile - nfull * CH

    def body(i, _):
        off = pl.multiple_of(base + i * CH, 8)
        pltpu.sync_copy(ei_hbm.at[1, pl.ds(off, CH)], didx)
        pltpu.sync_copy(ones_v, acc.at[didx, :], add=True)
        return ()

    lax.fori_loop(0, nfull, body, (), unroll=False)
    off = pl.multiple_of(base + nfull * CH, 8)
    pltpu.sync_copy(ei_hbm.at[1, pl.ds(off, tail)], didx_t)
    pltpu.sync_copy(ones_v.at[pl.ds(0, tail), :], acc.at[didx_t, :], add=True)

    plsc.subcore_barrier()
    nw = 3128
    w0 = pl.multiple_of(s * nw, 8)
    last = NN - 15 * nw

    @pl.when(s < NS - 1)
    def _():
        pltpu.sync_copy(acc.at[pl.ds(w0, nw), :],
                        deg_out.at[c, pl.ds(w0, nw), :])

    @pl.when(s == NS - 1)
    def _():
        pltpu.sync_copy(acc.at[pl.ds(w0, last), :],
                        deg_out.at[c, pl.ds(w0, last), :])


_deg_kernel = functools.partial(
    pl.kernel,
    out_type=jax.ShapeDtypeStruct((NC, NN, 1), jnp.float32),
    mesh=_mesh,
    scratch_types=[
        pltpu.VMEM((CH,), jnp.int32),
        pltpu.VMEM((40,), jnp.int32),
        pltpu.VMEM((CH, 1), jnp.float32),
        pltpu.VMEM((3128, 1), jnp.float32),
        pltpu.VMEM_SHARED((DEG_WORDS, 1), jnp.float32),
    ],
)(_deg_body)


def _agg_body(g_hbm, ei_hbm, z_hbm, out_hbm, sidx, dbuf, sidx_t, dbuf_t,
              rows, rows_t, zv, acc, gsem):
    c = lax.axis_index("c")
    s = lax.axis_index("s")
    base_row = c * HALF

    pltpu.sync_copy(z_hbm, zv)
    r0 = s * 1568
    for k in range(4):
        pltpu.sync_copy(zv, acc.at[pl.ds(r0 + k * 392, 392), :])
    plsc.subcore_barrier()

    per_tile = EE // NS
    base = s * per_tile
    nfull = per_tile // CH
    tail = per_tile - nfull * CH

    def remap(ref, n):
        for j in range(n // LANES):
            dv = ref[pl.ds(j * LANES, LANES)]
            loc = dv - base_row
            ok = (loc >= 0) & (loc < HALF)
            ref[pl.ds(j * LANES, LANES)] = jnp.where(ok, loc, TRASH)

    def body(i, _):
        off = pl.multiple_of(base + i * CH, 8)
        pltpu.sync_copy(ei_hbm.at[0, pl.ds(off, CH)], sidx)
        pltpu.sync_copy(ei_hbm.at[1, pl.ds(off, CH)], dbuf)
        pltpu.async_copy(g_hbm.at[sidx], rows, gsem).wait()
        remap(dbuf, CH)
        pltpu.sync_copy(rows, acc.at[dbuf], add=True)
        return ()

    lax.fori_loop(0, nfull, body, (), unroll=False)
    off = pl.multiple_of(base + nfull * CH, 8)
    pltpu.sync_copy(ei_hbm.at[0, pl.ds(off, tail)], sidx_t)
    pltpu.sync_copy(ei_hbm.at[1, pl.ds(off, tail)], dbuf_t)
    pltpu.async_copy(g_hbm.at[sidx_t], rows_t, gsem).wait()
    remap(dbuf_t, tail)
    pltpu.sync_copy(rows_t, acc.at[dbuf_t], add=True)

    plsc.subcore_barrier()
    stripe = 1568
    o0 = s * stripe
    last = HALF - 15 * stripe

    @pl.when(s < NS - 1)
    def _():
        pltpu.sync_copy(acc.at[pl.ds(o0, stripe), :],
                        out_hbm.at[pl.ds(base_row + o0, stripe), :])

    @pl.when(s == NS - 1)
    def _():
        pltpu.sync_copy(acc.at[pl.ds(o0, last), :],
                        out_hbm.at[pl.ds(base_row + o0, last), :])


_agg_kernel = functools.partial(
    pl.kernel,
    out_type=jax.ShapeDtypeStruct((NN, DD), jnp.float32),
    mesh=_mesh,
    scratch_types=[
        pltpu.VMEM((CH,), jnp.int32),
        pltpu.VMEM((CH,), jnp.int32),
        pltpu.VMEM((80,), jnp.int32),
        pltpu.VMEM((80,), jnp.int32),
        pltpu.VMEM((CH, DD), jnp.float32),
        pltpu.VMEM((80, DD), jnp.float32),
        pltpu.VMEM((392, DD), jnp.float32),
        pltpu.VMEM_SHARED((ACC_ROWS, DD), jnp.float32),
        pltpu.SemaphoreType.DMA,
    ],
)(_agg_body)


BLK = 1000
GRID = NN // BLK


def _b1_body(x_ref, w_ref, deg_ref, g_ref, dinv_ref):
    deg = deg_ref[0] + deg_ref[1] + 1.0
    dinv = lax.rsqrt(deg)
    dinv_ref[...] = dinv
    g_ref[...] = jnp.dot(x_ref[...], w_ref[...],
                         preferred_element_type=jnp.float32) * dinv


def _b1(x, W0, deg2):
    return pl.pallas_call(
        _b1_body,
        grid=(GRID,),
        in_specs=[
            pl.BlockSpec((BLK, DD), lambda i: (i, 0)),
            pl.BlockSpec((DD, DD), lambda i: (0, 0)),
            pl.BlockSpec((NC, BLK, 1), lambda i: (0, i, 0)),
        ],
        out_specs=[
            pl.BlockSpec((BLK, DD), lambda i: (i, 0)),
            pl.BlockSpec((BLK, 1), lambda i: (i, 0)),
        ],
        out_shape=[
            jax.ShapeDtypeStruct((NN, DD), jnp.float32),
            jax.ShapeDtypeStruct((NN, 1), jnp.float32),
        ],
    )(x, W0, deg2)


def _b2_body(raw_ref, g_ref, dinv_ref, w_ref, b_ref, out_ref):
    dinv = dinv_ref[...]
    h1 = (raw_ref[...] + g_ref[...]) * dinv + b_ref[...]
    out_ref[...] = jnp.dot(h1, w_ref[...],
                           preferred_element_type=jnp.float32) * dinv


def _b2(raw1, g1, dinv, W1, b0):
    return pl.pallas_call(
        _b2_body,
        grid=(GRID,),
        in_specs=[
            pl.BlockSpec((BLK, DD), lambda i: (i, 0)),
            pl.BlockSpec((BLK, DD), lambda i: (i, 0)),
            pl.BlockSpec((BLK, 1), lambda i: (i, 0)),
            pl.BlockSpec((DD, DD), lambda i: (0, 0)),
            pl.BlockSpec((1, DD), lambda i: (0, 0)),
        ],
        out_specs=pl.BlockSpec((BLK, DD), lambda i: (i, 0)),
        out_shape=jax.ShapeDtypeStruct((NN, DD), jnp.float32),
    )(raw1, g1, dinv, W1, b0)


def _d1_body(raw_ref, g_ref, dinv_ref, b_ref, h_ref, stat_ref, acc_ref):
    i = pl.program_id(0)
    h = (raw_ref[...] + g_ref[...]) * dinv_ref[...] + b_ref[...]
    h_ref[...] = h
    ps = jnp.sum(h, 0, keepdims=True)
    pq = jnp.sum(h * h, 0, keepdims=True)
    blk = jnp.concatenate([ps, pq], 0)

    @pl.when(i == 0)
    def _():
        acc_ref[...] = blk

    @pl.when(i > 0)
    def _():
        acc_ref[...] += blk

    @pl.when(i == pl.num_programs(0) - 1)
    def _():
        stat_ref[...] = acc_ref[...]


def _d1(raw2, g2, dinv, b1):
    return pl.pallas_call(
        _d1_body,
        grid=(GRID,),
        in_specs=[
            pl.BlockSpec((BLK, DD), lambda i: (i, 0)),
            pl.BlockSpec((BLK, DD), lambda i: (i, 0)),
            pl.BlockSpec((BLK, 1), lambda i: (i, 0)),
            pl.BlockSpec((1, DD), lambda i: (0, 0)),
        ],
        out_specs=[
            pl.BlockSpec((BLK, DD), lambda i: (i, 0)),
            pl.BlockSpec((2, DD), lambda i: (0, 0)),
        ],
        out_shape=[
            jax.ShapeDtypeStruct((NN, DD), jnp.float32),
            jax.ShapeDtypeStruct((2, DD), jnp.float32),
        ],
        scratch_shapes=[pltpu.VMEM((2, DD), jnp.float32)],
    )(raw2, g2, dinv, b1)


def _d2_body(h_ref, stat_ref, gamma_ref, beta_ref, out_ref):
    mean = stat_ref[0:1] * (1.0 / NN)
    var = stat_ref[1:2] * (1.0 / NN) - mean * mean
    rstd = lax.rsqrt(var + 1e-5)
    out_ref[...] = (h_ref[...] - mean) * rstd * gamma_ref[...] + beta_ref[...]


def _d2(h2, stat, gamma, beta):
    return pl.pallas_call(
        _d2_body,
        grid=(GRID,),
        in_specs=[
            pl.BlockSpec((BLK, DD), lambda i: (i, 0)),
            pl.BlockSpec((2, DD), lambda i: (0, 0)),
            pl.BlockSpec((1, DD), lambda i: (0, 0)),
            pl.BlockSpec((1, DD), lambda i: (0, 0)),
        ],
        out_specs=pl.BlockSpec((BLK, DD), lambda i: (i, 0)),
        out_shape=jax.ShapeDtypeStruct((NN, DD), jnp.float32),
    )(h2, stat, gamma, beta)


def kernel(x, edge_index, W0, b0, W1, b1, gamma, beta):
    ei = edge_index.astype(jnp.int32)
    ones = jnp.ones((CH, 1), jnp.float32)
    z1 = jnp.zeros((3128, 1), jnp.float32)
    zrows = jnp.zeros((392, DD), jnp.float32)

    deg2 = _deg_kernel(ei, ones, z1)
    g1, dinv = _b1(x, W0, deg2)
    raw1 = _agg_kernel(g1, ei, zrows)
    g2 = _b2(raw1, g1, dinv, W1, b0.reshape(1, DD))
    raw2 = _agg_kernel(g2, ei, zrows)
    h2, stat = _d1(raw2, g2, dinv, b1.reshape(1, DD))
    out = _d2(h2, stat, gamma.reshape(1, DD), beta.reshape(1, DD))
    return (out[:NUSERS], out[NUSERS:])


# speedup vs baseline: 6.7997x; 6.7997x over previous
import functools

import jax
import jax.numpy as jnp
from jax import lax
from jax.experimental import pallas as pl
from jax.experimental.pallas import tpu as pltpu
from jax.experimental.pallas import tpu_sc as plsc

NUSERS = 20000
NN = 50000
DD = 64
EE = 800000
NC, NS, LANES = 2, 16, 16
HALF = NN // 2
TRASH = HALF
ACC_ROWS = 16 * 1564
DEG_WORDS = 16 * 3128
DH = DD // 2

CH = 128

_mesh = plsc.VectorSubcoreMesh(
    core_axis_name="c", subcore_axis_name="s", num_cores=NC, num_subcores=NS)
_sc_params = pltpu.CompilerParams(use_tc_tiling_on_sc=False)


def _deg_body(dst_hbm, ones_hbm, z_hbm, deg_out, didx, didx_t, ones_v, zv,
              acc):
    c = lax.axis_index("c")
    s = lax.axis_index("s")
    pltpu.sync_copy(z_hbm, zv)
    pltpu.sync_copy(zv, acc.at[pl.ds(pl.multiple_of(s * 3128, 8), 3128), :])
    pltpu.sync_copy(ones_hbm, ones_v)
    plsc.subcore_barrier()

    per_tile = EE // (NC * NS)
    base = (c * NS + s) * per_tile
    nfull = per_tile // CH
    tail = per_tile - nfull * CH

    def body(i, _):
        off = pl.multiple_of(base + i * CH, 8)
        pltpu.sync_copy(dst_hbm.at[pl.ds(off, CH)], didx)
        pltpu.sync_copy(ones_v, acc.at[didx, :], add=True)
        return ()

    lax.fori_loop(0, nfull, body, (), unroll=False)
    off = pl.multiple_of(base + nfull * CH, 8)
    pltpu.sync_copy(dst_hbm.at[pl.ds(off, tail)], didx_t)
    pltpu.sync_copy(ones_v.at[pl.ds(0, tail), :], acc.at[didx_t, :], add=True)

    plsc.subcore_barrier()
    nw = 3128
    w0 = pl.multiple_of(s * nw, 8)
    last = NN - 15 * nw

    @pl.when(s < NS - 1)
    def _():
        pltpu.sync_copy(acc.at[pl.ds(w0, nw), :],
                        deg_out.at[c, pl.ds(w0, nw), :])

    @pl.when(s == NS - 1)
    def _():
        pltpu.sync_copy(acc.at[pl.ds(w0, last), :],
                        deg_out.at[c, pl.ds(w0, last), :])


_deg_kernel = functools.partial(
    pl.kernel,
    out_type=jax.ShapeDtypeStruct((NC, NN, 16), jnp.float32),
    mesh=_mesh,
    scratch_types=[
        pltpu.VMEM((CH,), jnp.int32),
        pltpu.VMEM((40,), jnp.int32),
        pltpu.VMEM((CH, 16), jnp.float32),
        pltpu.VMEM((3128, 16), jnp.float32),
        pltpu.VMEM_SHARED((DEG_WORDS, 16), jnp.float32),
    ],
    compiler_params=_sc_params,
)(_deg_body)


def _agg_body(ga_hbm, gb_hbm, src_hbm, dst_hbm, z_hbm, outa_hbm, outb_hbm,
              sidx, dbuf, sidx_t, dbuf_t, rows, rows_t, zv, acc, gsem):
    c = lax.axis_index("c")
    s = lax.axis_index("s")
    base_row = c * HALF

    per_tile = EE // NS
    base = s * per_tile
    nfull = per_tile // CH
    tail = per_tile - nfull * CH
    stripe = 1564
    r0 = s * stripe
    last = HALF - 15 * stripe

    def remap(ref, n):
        for j in range(n // LANES):
            dv = ref[pl.ds(j * LANES, LANES)]
            loc = dv - base_row
            ok = (loc >= 0) & (loc < HALF)
            ref[pl.ds(j * LANES, LANES)] = jnp.where(ok, loc, TRASH)

    pltpu.sync_copy(z_hbm, zv)

    for tab, out_hbm in ((ga_hbm, outa_hbm), (gb_hbm, outb_hbm)):
        for k in range(4):
            pltpu.sync_copy(zv, acc.at[pl.ds(r0 + k * 391, 391), :])
        plsc.subcore_barrier()

        def body(i, _):
            off = pl.multiple_of(base + i * CH, 8)
            pltpu.sync_copy(src_hbm.at[pl.ds(off, CH)], sidx)
            pltpu.sync_copy(dst_hbm.at[pl.ds(off, CH)], dbuf)
            pltpu.async_copy(tab.at[sidx], rows, gsem).wait()
            remap(dbuf, CH)
            pltpu.sync_copy(rows, acc.at[dbuf], add=True)
            return ()

        lax.fori_loop(0, nfull, body, (), unroll=False)
        off = pl.multiple_of(base + nfull * CH, 8)
        pltpu.sync_copy(src_hbm.at[pl.ds(off, tail)], sidx_t)
        pltpu.sync_copy(dst_hbm.at[pl.ds(off, tail)], dbuf_t)
        pltpu.async_copy(tab.at[sidx_t], rows_t, gsem).wait()
        remap(dbuf_t, tail)
        pltpu.sync_copy(rows_t, acc.at[dbuf_t], add=True)

        plsc.subcore_barrier()
        @pl.when(s < NS - 1)
        def _():
            pltpu.sync_copy(acc.at[pl.ds(r0, stripe), :],
                            out_hbm.at[pl.ds(base_row + r0, stripe), :])

        @pl.when(s == NS - 1)
        def _():
            pltpu.sync_copy(acc.at[pl.ds(r0, last), :],
                            out_hbm.at[pl.ds(base_row + r0, last), :])


_agg_kernel = functools.partial(
    pl.kernel,
    out_type=[jax.ShapeDtypeStruct((NN, DH), jnp.float32),
              jax.ShapeDtypeStruct((NN, DH), jnp.float32)],
    mesh=_mesh,
    scratch_types=[
        pltpu.VMEM((CH,), jnp.int32),
        pltpu.VMEM((CH,), jnp.int32),
        pltpu.VMEM((80,), jnp.int32),
        pltpu.VMEM((80,), jnp.int32),
        pltpu.VMEM((CH, DH), jnp.float32),
        pltpu.VMEM((80, DH), jnp.float32),
        pltpu.VMEM((391, DH), jnp.float32),
        pltpu.VMEM_SHARED((ACC_ROWS, DH), jnp.float32),
        pltpu.SemaphoreType.DMA,
    ],
    compiler_params=_sc_params,
)(_agg_body)


BLK = 1000
GRID = NN // BLK


def _b1_body(x_ref, w_ref, deg_ref, ga_ref, gb_ref, dinv_ref):
    deg = deg_ref[0, :, 0:1] + deg_ref[1, :, 0:1] + 1.0
    dinv = lax.rsqrt(deg)
    dinv_ref[...] = dinv
    g = jnp.dot(x_ref[...], w_ref[...],
                preferred_element_type=jnp.float32) * dinv
    ga_ref[...] = g[:, :DH]
    gb_ref[...] = g[:, DH:]


def _b1(x, W0, deg2):
    return pl.pallas_call(
        _b1_body,
        grid=(GRID,),
        in_specs=[
            pl.BlockSpec((BLK, DD), lambda i: (i, 0)),
            pl.BlockSpec((DD, DD), lambda i: (0, 0)),
            pl.BlockSpec((NC, BLK, 16), lambda i: (0, i, 0)),
        ],
        out_specs=[
            pl.BlockSpec((BLK, DH), lambda i: (i, 0)),
            pl.BlockSpec((BLK, DH), lambda i: (i, 0)),
            pl.BlockSpec((BLK, 1), lambda i: (i, 0)),
        ],
        out_shape=[
            jax.ShapeDtypeStruct((NN, DH), jnp.float32),
            jax.ShapeDtypeStruct((NN, DH), jnp.float32),
            jax.ShapeDtypeStruct((NN, 1), jnp.float32),
        ],
    )(x, W0, deg2)


def _b2_body(ra_ref, rb_ref, ga_ref, gb_ref, dinv_ref, w_ref, b_ref,
             oa_ref, ob_ref):
    dinv = dinv_ref[...]
    ha = (ra_ref[...] + ga_ref[...]) * dinv + b_ref[:, :DH]
    hb = (rb_ref[...] + gb_ref[...]) * dinv + b_ref[:, DH:]
    h1 = jnp.concatenate([ha, hb], 1)
    g2 = jnp.dot(h1, w_ref[...], preferred_element_type=jnp.float32) * dinv
    oa_ref[...] = g2[:, :DH]
    ob_ref[...] = g2[:, DH:]


def _b2(ra, rb, ga, gb, dinv, W1, b0):
    return pl.pallas_call(
        _b2_body,
        grid=(GRID,),
        in_specs=[
            pl.BlockSpec((BLK, DH), lambda i: (i, 0)),
            pl.BlockSpec((BLK, DH), lambda i: (i, 0)),
            pl.BlockSpec((BLK, DH), lambda i: (i, 0)),
            pl.BlockSpec((BLK, DH), lambda i: (i, 0)),
            pl.BlockSpec((BLK, 1), lambda i: (i, 0)),
            pl.BlockSpec((DD, DD), lambda i: (0, 0)),
            pl.BlockSpec((1, DD), lambda i: (0, 0)),
        ],
        out_specs=[
            pl.BlockSpec((BLK, DH), lambda i: (i, 0)),
            pl.BlockSpec((BLK, DH), lambda i: (i, 0)),
        ],
        out_shape=[
            jax.ShapeDtypeStruct((NN, DH), jnp.float32),
            jax.ShapeDtypeStruct((NN, DH), jnp.float32),
        ],
    )(ra, rb, ga, gb, dinv, W1, b0)


def _d1_body(ra_ref, rb_ref, ga_ref, gb_ref, dinv_ref, b_ref, h_ref,
             stat_ref, acc_ref):
    i = pl.program_id(0)
    dinv = dinv_ref[...]
    ha = (ra_ref[...] + ga_ref[...]) * dinv + b_ref[:, :DH]
    hb = (rb_ref[...] + gb_ref[...]) * dinv + b_ref[:, DH:]
    h = jnp.concatenate([ha, hb], 1)
    h_ref[...] = h
    ps = jnp.sum(h, 0, keepdims=True)
    pq = jnp.sum(h * h, 0, keepdims=True)
    blk = jnp.concatenate([ps, pq], 0)

    @pl.when(i == 0)
    def _():
        acc_ref[...] = blk

    @pl.when(i > 0)
    def _():
        acc_ref[...] += blk

    @pl.when(i == pl.num_programs(0) - 1)
    def _():
        stat_ref[...] = acc_ref[...]


def _d1(r2a, r2b, g2a, g2b, dinv, b1):
    return pl.pallas_call(
        _d1_body,
        grid=(GRID,),
        in_specs=[
            pl.BlockSpec((BLK, DH), lambda i: (i, 0)),
            pl.BlockSpec((BLK, DH), lambda i: (i, 0)),
            pl.BlockSpec((BLK, DH), lambda i: (i, 0)),
            pl.BlockSpec((BLK, DH), lambda i: (i, 0)),
            pl.BlockSpec((BLK, 1), lambda i: (i, 0)),
            pl.BlockSpec((1, DD), lambda i: (0, 0)),
        ],
        out_specs=[
            pl.BlockSpec((BLK, DD), lambda i: (i, 0)),
            pl.BlockSpec((2, DD), lambda i: (0, 0)),
        ],
        out_shape=[
            jax.ShapeDtypeStruct((NN, DD), jnp.float32),
            jax.ShapeDtypeStruct((2, DD), jnp.float32),
        ],
        scratch_shapes=[pltpu.VMEM((2, DD), jnp.float32)],
    )(r2a, r2b, g2a, g2b, dinv, b1)


def _d2_body(h_ref, stat_ref, gamma_ref, beta_ref, out_ref):
    mean = stat_ref[0:1] * (1.0 / NN)
    var = stat_ref[1:2] * (1.0 / NN) - mean * mean
    rstd = lax.rsqrt(var + 1e-5)
    out_ref[...] = (h_ref[...] - mean) * rstd * gamma_ref[...] + beta_ref[...]


def _d2(h2, stat, gamma, beta):
    return pl.pallas_call(
        _d2_body,
        grid=(GRID,),
        in_specs=[
            pl.BlockSpec((BLK, DD), lambda i: (i, 0)),
            pl.BlockSpec((2, DD), lambda i: (0, 0)),
            pl.BlockSpec((1, DD), lambda i: (0, 0)),
            pl.BlockSpec((1, DD), lambda i: (0, 0)),
        ],
        out_specs=pl.BlockSpec((BLK, DD), lambda i: (i, 0)),
        out_shape=jax.ShapeDtypeStruct((NN, DD), jnp.float32),
    )(h2, stat, gamma, beta)


def kernel(x, edge_index, W0, b0, W1, b1, gamma, beta):
    ei = edge_index.astype(jnp.int32)
    src, dst = ei[0], ei[1]
    ones = jnp.ones((CH, 16), jnp.float32)
    z1 = jnp.zeros((3128, 16), jnp.float32)
    zrows = jnp.zeros((391, DH), jnp.float32)

    deg2 = _deg_kernel(dst, ones, z1)
    ga, gb, dinv = _b1(x, W0, deg2)
    ra, rb = _agg_kernel(ga, gb, src, dst, zrows)
    g2a, g2b = _b2(ra, rb, ga, gb, dinv, W1, b0.reshape(1, DD))
    r2a, r2b = _agg_kernel(g2a, g2b, src, dst, zrows)
    h2, stat = _d1(r2a, r2b, g2a, g2b, dinv, b1.reshape(1, DD))
    out = _d2(h2, stat, gamma.reshape(1, DD), beta.reshape(1, DD))
    return (out[:NUSERS], out[NUSERS:])


# speedup vs baseline: 9.6348x; 1.4169x over previous
import functools

import jax
import jax.numpy as jnp
from jax import lax
from jax.experimental import pallas as pl
from jax.experimental.pallas import tpu as pltpu
from jax.experimental.pallas import tpu_sc as plsc

NUSERS = 20000
NN = 50000
DD = 64
EE = 800000
NC, NS, LANES = 2, 16, 16
HALF = NN // 2
TRASH = HALF
ACC_ROWS = 16 * 1564
DEG_WORDS = 16 * 3128
DH = DD // 2

CH = 128
CHA = 512
TAILA = 336

_mesh = plsc.VectorSubcoreMesh(
    core_axis_name="c", subcore_axis_name="s", num_cores=NC, num_subcores=NS)
_sc_params = pltpu.CompilerParams(use_tc_tiling_on_sc=False)


def _deg_body(dst_hbm, ones_hbm, z_hbm, deg_out, didx, didx_t, ones_v, zv,
              acc):
    c = lax.axis_index("c")
    s = lax.axis_index("s")
    pltpu.sync_copy(z_hbm, zv)
    pltpu.sync_copy(zv, acc.at[pl.ds(pl.multiple_of(s * 3128, 8), 3128), :])
    pltpu.sync_copy(ones_hbm, ones_v)
    plsc.subcore_barrier()

    per_tile = EE // (NC * NS)
    base = (c * NS + s) * per_tile
    nfull = per_tile // CH
    tail = per_tile - nfull * CH

    def body(i, _):
        off = pl.multiple_of(base + i * CH, 8)
        pltpu.sync_copy(dst_hbm.at[pl.ds(off, CH)], didx)
        pltpu.sync_copy(ones_v, acc.at[didx, :], add=True)
        return ()

    lax.fori_loop(0, nfull, body, (), unroll=False)
    off = pl.multiple_of(base + nfull * CH, 8)
    pltpu.sync_copy(dst_hbm.at[pl.ds(off, tail)], didx_t)
    pltpu.sync_copy(ones_v.at[pl.ds(0, tail), :], acc.at[didx_t, :], add=True)

    plsc.subcore_barrier()
    nw = 3128
    w0 = pl.multiple_of(s * nw, 8)
    last = NN - 15 * nw

    @pl.when(s < NS - 1)
    def _():
        pltpu.sync_copy(acc.at[pl.ds(w0, nw), :],
                        deg_out.at[c, pl.ds(w0, nw), :])

    @pl.when(s == NS - 1)
    def _():
        pltpu.sync_copy(acc.at[pl.ds(w0, last), :],
                        deg_out.at[c, pl.ds(w0, last), :])


_deg_kernel = functools.partial(
    pl.kernel,
    out_type=jax.ShapeDtypeStruct((NC, NN, 16), jnp.float32),
    mesh=_mesh,
    scratch_types=[
        pltpu.VMEM((CH,), jnp.int32),
        pltpu.VMEM((40,), jnp.int32),
        pltpu.VMEM((CH, 16), jnp.float32),
        pltpu.VMEM((3128, 16), jnp.float32),
        pltpu.VMEM_SHARED((DEG_WORDS, 16), jnp.float32),
    ],
    compiler_params=_sc_params,
)(_deg_body)


def _agg_body(ga_hbm, gb_hbm, src_hbm, dst_hbm, z_hbm, outa_hbm, outb_hbm,
              sidx, dbuf, sidx_t, dbuf_t, rows, rows_t, zv, acc, gsem):
    c = lax.axis_index("c")
    s = lax.axis_index("s")
    base_row = c * HALF

    per_tile = EE // NS
    base = s * per_tile
    nfull = per_tile // CHA
    tail = per_tile - nfull * CHA
    stripe = 1564
    r0 = s * stripe
    last = HALF - 15 * stripe

    def remap(ref, n):
        for j in range(n // LANES):
            dv = ref[pl.ds(j * LANES, LANES)]
            loc = dv - base_row
            ok = (loc >= 0) & (loc < HALF)
            ref[pl.ds(j * LANES, LANES)] = jnp.where(ok, loc, TRASH)

    pltpu.sync_copy(z_hbm, zv)

    for tab, out_hbm in ((ga_hbm, outa_hbm), (gb_hbm, outb_hbm)):
        for k in range(4):
            pltpu.sync_copy(zv, acc.at[pl.ds(r0 + k * 391, 391), :])
        plsc.subcore_barrier()

        def body(i, _):
            off = pl.multiple_of(base + i * CHA, 8)
            pltpu.sync_copy(src_hbm.at[pl.ds(off, CHA)], sidx)
            pltpu.sync_copy(dst_hbm.at[pl.ds(off, CHA)], dbuf)
            pltpu.async_copy(tab.at[sidx], rows, gsem).wait()
            remap(dbuf, CHA)
            pltpu.sync_copy(rows, acc.at[dbuf], add=True)
            return ()

        lax.fori_loop(0, nfull, body, (), unroll=False)
        off = pl.multiple_of(base + nfull * CHA, 8)
        pltpu.sync_copy(src_hbm.at[pl.ds(off, tail)], sidx_t)
        pltpu.sync_copy(dst_hbm.at[pl.ds(off, tail)], dbuf_t)
        pltpu.async_copy(tab.at[sidx_t], rows_t, gsem).wait()
        remap(dbuf_t, tail)
        pltpu.sync_copy(rows_t, acc.at[dbuf_t], add=True)

        plsc.subcore_barrier()
        @pl.when(s < NS - 1)
        def _():
            pltpu.sync_copy(acc.at[pl.ds(r0, stripe), :],
                            out_hbm.at[pl.ds(base_row + r0, stripe), :])

        @pl.when(s == NS - 1)
        def _():
            pltpu.sync_copy(acc.at[pl.ds(r0, last), :],
                            out_hbm.at[pl.ds(base_row + r0, last), :])


_agg_kernel = functools.partial(
    pl.kernel,
    out_type=[jax.ShapeDtypeStruct((NN, DH), jnp.float32),
              jax.ShapeDtypeStruct((NN, DH), jnp.float32)],
    mesh=_mesh,
    scratch_types=[
        pltpu.VMEM((CHA,), jnp.int32),
        pltpu.VMEM((CHA,), jnp.int32),
        pltpu.VMEM((TAILA,), jnp.int32),
        pltpu.VMEM((TAILA,), jnp.int32),
        pltpu.VMEM((CHA, DH), jnp.float32),
        pltpu.VMEM((TAILA, DH), jnp.float32),
        pltpu.VMEM((391, DH), jnp.float32),
        pltpu.VMEM_SHARED((ACC_ROWS, DH), jnp.float32),
        pltpu.SemaphoreType.DMA,
    ],
    compiler_params=_sc_params,
)(_agg_body)


BLK = 1000
GRID = NN // BLK


def _b1_body(x_ref, w_ref, deg_ref, ga_ref, gb_ref, dinv_ref):
    deg = deg_ref[0, :, 0:1] + deg_ref[1, :, 0:1] + 1.0
    dinv = lax.rsqrt(deg)
    dinv_ref[...] = dinv
    g = jnp.dot(x_ref[...], w_ref[...],
                preferred_element_type=jnp.float32) * dinv
    ga_ref[...] = g[:, :DH]
    gb_ref[...] = g[:, DH:]


def _b1(x, W0, deg2):
    return pl.pallas_call(
        _b1_body,
        grid=(GRID,),
        in_specs=[
            pl.BlockSpec((BLK, DD), lambda i: (i, 0)),
            pl.BlockSpec((DD, DD), lambda i: (0, 0)),
            pl.BlockSpec((NC, BLK, 16), lambda i: (0, i, 0)),
        ],
        out_specs=[
            pl.BlockSpec((BLK, DH), lambda i: (i, 0)),
            pl.BlockSpec((BLK, DH), lambda i: (i, 0)),
            pl.BlockSpec((BLK, 1), lambda i: (i, 0)),
        ],
        out_shape=[
            jax.ShapeDtypeStruct((NN, DH), jnp.float32),
            jax.ShapeDtypeStruct((NN, DH), jnp.float32),
            jax.ShapeDtypeStruct((NN, 1), jnp.float32),
        ],
    )(x, W0, deg2)


def _b2_body(ra_ref, rb_ref, ga_ref, gb_ref, dinv_ref, w_ref, b_ref,
             oa_ref, ob_ref):
    dinv = dinv_ref[...]
    ha = (ra_ref[...] + ga_ref[...]) * dinv + b_ref[:, :DH]
    hb = (rb_ref[...] + gb_ref[...]) * dinv + b_ref[:, DH:]
    h1 = jnp.concatenate([ha, hb], 1)
    g2 = jnp.dot(h1, w_ref[...], preferred_element_type=jnp.float32) * dinv
    oa_ref[...] = g2[:, :DH]
    ob_ref[...] = g2[:, DH:]


def _b2(ra, rb, ga, gb, dinv, W1, b0):
    return pl.pallas_call(
        _b2_body,
        grid=(GRID,),
        in_specs=[
            pl.BlockSpec((BLK, DH), lambda i: (i, 0)),
            pl.BlockSpec((BLK, DH), lambda i: (i, 0)),
            pl.BlockSpec((BLK, DH), lambda i: (i, 0)),
            pl.BlockSpec((BLK, DH), lambda i: (i, 0)),
            pl.BlockSpec((BLK, 1), lambda i: (i, 0)),
            pl.BlockSpec((DD, DD), lambda i: (0, 0)),
            pl.BlockSpec((1, DD), lambda i: (0, 0)),
        ],
        out_specs=[
            pl.BlockSpec((BLK, DH), lambda i: (i, 0)),
            pl.BlockSpec((BLK, DH), lambda i: (i, 0)),
        ],
        out_shape=[
            jax.ShapeDtypeStruct((NN, DH), jnp.float32),
            jax.ShapeDtypeStruct((NN, DH), jnp.float32),
        ],
    )(ra, rb, ga, gb, dinv, W1, b0)


def _d1_body(ra_ref, rb_ref, ga_ref, gb_ref, dinv_ref, b_ref, h_ref,
             stat_ref, acc_ref):
    i = pl.program_id(0)
    dinv = dinv_ref[...]
    ha = (ra_ref[...] + ga_ref[...]) * dinv + b_ref[:, :DH]
    hb = (rb_ref[...] + gb_ref[...]) * dinv + b_ref[:, DH:]
    h = jnp.concatenate([ha, hb], 1)
    h_ref[...] = h
    ps = jnp.sum(h, 0, keepdims=True)
    pq = jnp.sum(h * h, 0, keepdims=True)
    blk = jnp.concatenate([ps, pq], 0)

    @pl.when(i == 0)
    def _():
        acc_ref[...] = blk

    @pl.when(i > 0)
    def _():
        acc_ref[...] += blk

    @pl.when(i == pl.num_programs(0) - 1)
    def _():
        stat_ref[...] = acc_ref[...]


def _d1(r2a, r2b, g2a, g2b, dinv, b1):
    return pl.pallas_call(
        _d1_body,
        grid=(GRID,),
        in_specs=[
            pl.BlockSpec((BLK, DH), lambda i: (i, 0)),
            pl.BlockSpec((BLK, DH), lambda i: (i, 0)),
            pl.BlockSpec((BLK, DH), lambda i: (i, 0)),
            pl.BlockSpec((BLK, DH), lambda i: (i, 0)),
            pl.BlockSpec((BLK, 1), lambda i: (i, 0)),
            pl.BlockSpec((1, DD), lambda i: (0, 0)),
        ],
        out_specs=[
            pl.BlockSpec((BLK, DD), lambda i: (i, 0)),
            pl.BlockSpec((2, DD), lambda i: (0, 0)),
        ],
        out_shape=[
            jax.ShapeDtypeStruct((NN, DD), jnp.float32),
            jax.ShapeDtypeStruct((2, DD), jnp.float32),
        ],
        scratch_shapes=[pltpu.VMEM((2, DD), jnp.float32)],
    )(r2a, r2b, g2a, g2b, dinv, b1)


def _d2_body(h_ref, stat_ref, gamma_ref, beta_ref, out_ref):
    mean = stat_ref[0:1] * (1.0 / NN)
    var = stat_ref[1:2] * (1.0 / NN) - mean * mean
    rstd = lax.rsqrt(var + 1e-5)
    out_ref[...] = (h_ref[...] - mean) * rstd * gamma_ref[...] + beta_ref[...]


def _d2(h2, stat, gamma, beta):
    return pl.pallas_call(
        _d2_body,
        grid=(GRID,),
        in_specs=[
            pl.BlockSpec((BLK, DD), lambda i: (i, 0)),
            pl.BlockSpec((2, DD), lambda i: (0, 0)),
            pl.BlockSpec((1, DD), lambda i: (0, 0)),
            pl.BlockSpec((1, DD), lambda i: (0, 0)),
        ],
        out_specs=pl.BlockSpec((BLK, DD), lambda i: (i, 0)),
        out_shape=jax.ShapeDtypeStruct((NN, DD), jnp.float32),
    )(h2, stat, gamma, beta)


def kernel(x, edge_index, W0, b0, W1, b1, gamma, beta):
    ei = edge_index.astype(jnp.int32)
    src, dst = ei[0], ei[1]
    ones = jnp.ones((CH, 16), jnp.float32)
    z1 = jnp.zeros((3128, 16), jnp.float32)
    zrows = jnp.zeros((391, DH), jnp.float32)

    deg2 = _deg_kernel(dst, ones, z1)
    ga, gb, dinv = _b1(x, W0, deg2)
    ra, rb = _agg_kernel(ga, gb, src, dst, zrows)
    g2a, g2b = _b2(ra, rb, ga, gb, dinv, W1, b0.reshape(1, DD))
    r2a, r2b = _agg_kernel(g2a, g2b, src, dst, zrows)
    h2, stat = _d1(r2a, r2b, g2a, g2b, dinv, b1.reshape(1, DD))
    out = _d2(h2, stat, gamma.reshape(1, DD), beta.reshape(1, DD))
    return (out[:NUSERS], out[NUSERS:])


# speedup vs baseline: 9.7028x; 1.0071x over previous
import functools

import jax
import jax.numpy as jnp
from jax import lax
from jax.experimental import pallas as pl
from jax.experimental.pallas import tpu as pltpu
from jax.experimental.pallas import tpu_sc as plsc

NUSERS = 20000
NN = 50000
DD = 64
EE = 800000
NC, NS, LANES = 2, 16, 16
HALF = NN // 2
TRASH = HALF
ACC_ROWS = 16 * 1564
DEG_WORDS = 16 * 3128
DH = DD // 2

CH = 128
CHA = 1024
NITA = (EE // NS) // CHA
TAILA = (EE // NS) - NITA * CHA

_mesh = plsc.VectorSubcoreMesh(
    core_axis_name="c", subcore_axis_name="s", num_cores=NC, num_subcores=NS)
_sc_params = pltpu.CompilerParams(use_tc_tiling_on_sc=False)


def _deg_body(dst_hbm, ones_hbm, z_hbm, deg_out, didx, didx_t, ones_v, zv,
              acc):
    c = lax.axis_index("c")
    s = lax.axis_index("s")
    pltpu.sync_copy(z_hbm, zv)
    pltpu.sync_copy(zv, acc.at[pl.ds(pl.multiple_of(s * 3128, 8), 3128), :])
    pltpu.sync_copy(ones_hbm, ones_v)
    plsc.subcore_barrier()

    per_tile = EE // (NC * NS)
    base = (c * NS + s) * per_tile
    nfull = per_tile // CH
    tail = per_tile - nfull * CH

    def body(i, _):
        off = pl.multiple_of(base + i * CH, 8)
        pltpu.sync_copy(dst_hbm.at[pl.ds(off, CH)], didx)
        pltpu.sync_copy(ones_v, acc.at[didx, :], add=True)
        return ()

    lax.fori_loop(0, nfull, body, (), unroll=False)
    off = pl.multiple_of(base + nfull * CH, 8)
    pltpu.sync_copy(dst_hbm.at[pl.ds(off, tail)], didx_t)
    pltpu.sync_copy(ones_v.at[pl.ds(0, tail), :], acc.at[didx_t, :], add=True)

    plsc.subcore_barrier()
    nw = 3128
    w0 = pl.multiple_of(s * nw, 8)
    last = NN - 15 * nw

    @pl.when(s < NS - 1)
    def _():
        pltpu.sync_copy(acc.at[pl.ds(w0, nw), :],
                        deg_out.at[c, pl.ds(w0, nw), :])

    @pl.when(s == NS - 1)
    def _():
        pltpu.sync_copy(acc.at[pl.ds(w0, last), :],
                        deg_out.at[c, pl.ds(w0, last), :])


_deg_kernel = functools.partial(
    pl.kernel,
    out_type=jax.ShapeDtypeStruct((NC, NN, 16), jnp.float32),
    mesh=_mesh,
    scratch_types=[
        pltpu.VMEM((CH,), jnp.int32),
        pltpu.VMEM((40,), jnp.int32),
        pltpu.VMEM((CH, 16), jnp.float32),
        pltpu.VMEM((3128, 16), jnp.float32),
        pltpu.VMEM_SHARED((DEG_WORDS, 16), jnp.float32),
    ],
    compiler_params=_sc_params,
)(_deg_body)


def _agg_body(ga_hbm, gb_hbm, src_hbm, dst_hbm, z_hbm, outa_hbm, outb_hbm,
              sidx0, dbuf0, sidx1, dbuf1, tidx_s, tidx_d, rows0, rows1, zv,
              acc, gsem0, gsem1, ssem0, ssem1):
    c = lax.axis_index("c")
    s = lax.axis_index("s")
    base_row = c * HALF

    per_tile = EE // NS
    base = s * per_tile
    stripe = 1564
    r0 = s * stripe
    last = HALF - 15 * stripe

    def remap(ref, n):
        for j in range(n // LANES):
            dv = ref[pl.ds(j * LANES, LANES)]
            loc = dv - base_row
            ok = (loc >= 0) & (loc < HALF)
            ref[pl.ds(j * LANES, LANES)] = jnp.where(ok, loc, TRASH)

    def stage(i, si, di):
        off = pl.multiple_of(base + i * CHA, 8)
        pltpu.sync_copy(src_hbm.at[pl.ds(off, CHA)], si)
        pltpu.sync_copy(dst_hbm.at[pl.ds(off, CHA)], di)

    pltpu.sync_copy(z_hbm, zv)

    for tab, out_hbm in ((ga_hbm, outa_hbm), (gb_hbm, outb_hbm)):
        for k in range(8):
            pltpu.sync_copy(zv, acc.at[pl.ds(r0 + k * 184, 184), :])
        pltpu.sync_copy(zv.at[pl.ds(0, 92), :],
                        acc.at[pl.ds(r0 + 1472, 92), :])
        plsc.subcore_barrier()

        stage(0, sidx0, dbuf0)
        pltpu.async_copy(tab.at[sidx0], rows0, gsem0)

        def pair(p, _):
            i2 = p * 2

            @pl.when(i2 > 0)
            def _():
                pltpu.make_async_copy(rows1, acc.at[dbuf1], ssem1).wait()
            stage(i2 + 1, sidx1, dbuf1)
            pltpu.async_copy(tab.at[sidx1], rows1, gsem1)

            pltpu.make_async_copy(tab.at[sidx0], rows0, gsem0).wait()
            remap(dbuf0, CHA)
            pltpu.async_copy(rows0, acc.at[dbuf0], ssem0, add=True)

            @pl.when(i2 + 2 < NITA)
            def _():
                pltpu.make_async_copy(rows0, acc.at[dbuf0], ssem0).wait()
                stage(i2 + 2, sidx0, dbuf0)
                pltpu.async_copy(tab.at[sidx0], rows0, gsem0)

            pltpu.make_async_copy(tab.at[sidx1], rows1, gsem1).wait()
            remap(dbuf1, CHA)
            pltpu.async_copy(rows1, acc.at[dbuf1], ssem1, add=True)
            return ()

        lax.fori_loop(0, NITA // 2, pair, (), unroll=False)
        pltpu.make_async_copy(rows0, acc.at[dbuf0], ssem0).wait()
        pltpu.make_async_copy(rows1, acc.at[dbuf1], ssem1).wait()

        off = pl.multiple_of(base + NITA * CHA, 8)
        pltpu.sync_copy(src_hbm.at[pl.ds(off, TAILA)], tidx_s)
        pltpu.sync_copy(dst_hbm.at[pl.ds(off, TAILA)], tidx_d)
        pltpu.async_copy(tab.at[tidx_s], rows0.at[pl.ds(0, TAILA), :],
                         gsem0).wait()
        remap(tidx_d, TAILA)
        pltpu.sync_copy(rows0.at[pl.ds(0, TAILA), :], acc.at[tidx_d],
                        add=True)

        plsc.subcore_barrier()
        @pl.when(s < NS - 1)
        def _():
            pltpu.sync_copy(acc.at[pl.ds(r0, stripe), :],
                            out_hbm.at[pl.ds(base_row + r0, stripe), :])

        @pl.when(s == NS - 1)
        def _():
            pltpu.sync_copy(acc.at[pl.ds(r0, last), :],
                            out_hbm.at[pl.ds(base_row + r0, last), :])


_agg_kernel = functools.partial(
    pl.kernel,
    out_type=[jax.ShapeDtypeStruct((NN, DH), jnp.float32),
              jax.ShapeDtypeStruct((NN, DH), jnp.float32)],
    mesh=_mesh,
    scratch_types=[
        pltpu.VMEM((CHA,), jnp.int32),
        pltpu.VMEM((CHA,), jnp.int32),
        pltpu.VMEM((CHA,), jnp.int32),
        pltpu.VMEM((CHA,), jnp.int32),
        pltpu.VMEM((TAILA,), jnp.int32),
        pltpu.VMEM((TAILA,), jnp.int32),
        pltpu.VMEM((CHA, DH), jnp.float32),
        pltpu.VMEM((CHA, DH), jnp.float32),
        pltpu.VMEM((184, DH), jnp.float32),
        pltpu.VMEM_SHARED((ACC_ROWS, DH), jnp.float32),
        pltpu.SemaphoreType.DMA,
        pltpu.SemaphoreType.DMA,
        pltpu.SemaphoreType.DMA,
        pltpu.SemaphoreType.DMA,
    ],
    compiler_params=_sc_params,
)(_agg_body)


BLK = 1000
GRID = NN // BLK


def _b1_body(x_ref, w_ref, deg_ref, ga_ref, gb_ref, dinv_ref):
    deg = deg_ref[0, :, 0:1] + deg_ref[1, :, 0:1] + 1.0
    dinv = lax.rsqrt(deg)
    dinv_ref[...] = dinv
    g = jnp.dot(x_ref[...], w_ref[...],
                preferred_element_type=jnp.float32) * dinv
    ga_ref[...] = g[:, :DH]
    gb_ref[...] = g[:, DH:]


def _b1(x, W0, deg2):
    return pl.pallas_call(
        _b1_body,
        grid=(GRID,),
        in_specs=[
            pl.BlockSpec((BLK, DD), lambda i: (i, 0)),
            pl.BlockSpec((DD, DD), lambda i: (0, 0)),
            pl.BlockSpec((NC, BLK, 16), lambda i: (0, i, 0)),
        ],
        out_specs=[
            pl.BlockSpec((BLK, DH), lambda i: (i, 0)),
            pl.BlockSpec((BLK, DH), lambda i: (i, 0)),
            pl.BlockSpec((BLK, 1), lambda i: (i, 0)),
        ],
        out_shape=[
            jax.ShapeDtypeStruct((NN, DH), jnp.float32),
            jax.ShapeDtypeStruct((NN, DH), jnp.float32),
            jax.ShapeDtypeStruct((NN, 1), jnp.float32),
        ],
    )(x, W0, deg2)


def _b2_body(ra_ref, rb_ref, ga_ref, gb_ref, dinv_ref, w_ref, b_ref,
             oa_ref, ob_ref):
    dinv = dinv_ref[...]
    ha = (ra_ref[...] + ga_ref[...]) * dinv + b_ref[:, :DH]
    hb = (rb_ref[...] + gb_ref[...]) * dinv + b_ref[:, DH:]
    h1 = jnp.concatenate([ha, hb], 1)
    g2 = jnp.dot(h1, w_ref[...], preferred_element_type=jnp.float32) * dinv
    oa_ref[...] = g2[:, :DH]
    ob_ref[...] = g2[:, DH:]


def _b2(ra, rb, ga, gb, dinv, W1, b0):
    return pl.pallas_call(
        _b2_body,
        grid=(GRID,),
        in_specs=[
            pl.BlockSpec((BLK, DH), lambda i: (i, 0)),
            pl.BlockSpec((BLK, DH), lambda i: (i, 0)),
            pl.BlockSpec((BLK, DH), lambda i: (i, 0)),
            pl.BlockSpec((BLK, DH), lambda i: (i, 0)),
            pl.BlockSpec((BLK, 1), lambda i: (i, 0)),
            pl.BlockSpec((DD, DD), lambda i: (0, 0)),
            pl.BlockSpec((1, DD), lambda i: (0, 0)),
        ],
        out_specs=[
            pl.BlockSpec((BLK, DH), lambda i: (i, 0)),
            pl.BlockSpec((BLK, DH), lambda i: (i, 0)),
        ],
        out_shape=[
            jax.ShapeDtypeStruct((NN, DH), jnp.float32),
            jax.ShapeDtypeStruct((NN, DH), jnp.float32),
        ],
    )(ra, rb, ga, gb, dinv, W1, b0)


def _d1_body(ra_ref, rb_ref, ga_ref, gb_ref, dinv_ref, b_ref, h_ref,
             stat_ref, acc_ref):
    i = pl.program_id(0)
    dinv = dinv_ref[...]
    ha = (ra_ref[...] + ga_ref[...]) * dinv + b_ref[:, :DH]
    hb = (rb_ref[...] + gb_ref[...]) * dinv + b_ref[:, DH:]
    h = jnp.concatenate([ha, hb], 1)
    h_ref[...] = h
    ps = jnp.sum(h, 0, keepdims=True)
    pq = jnp.sum(h * h, 0, keepdims=True)
    blk = jnp.concatenate([ps, pq], 0)

    @pl.when(i == 0)
    def _():
        acc_ref[...] = blk

    @pl.when(i > 0)
    def _():
        acc_ref[...] += blk

    @pl.when(i == pl.num_programs(0) - 1)
    def _():
        stat_ref[...] = acc_ref[...]


def _d1(r2a, r2b, g2a, g2b, dinv, b1):
    return pl.pallas_call(
        _d1_body,
        grid=(GRID,),
        in_specs=[
            pl.BlockSpec((BLK, DH), lambda i: (i, 0)),
            pl.BlockSpec((BLK, DH), lambda i: (i, 0)),
            pl.BlockSpec((BLK, DH), lambda i: (i, 0)),
            pl.BlockSpec((BLK, DH), lambda i: (i, 0)),
            pl.BlockSpec((BLK, 1), lambda i: (i, 0)),
            pl.BlockSpec((1, DD), lambda i: (0, 0)),
        ],
        out_specs=[
            pl.BlockSpec((BLK, DD), lambda i: (i, 0)),
            pl.BlockSpec((2, DD), lambda i: (0, 0)),
        ],
        out_shape=[
            jax.ShapeDtypeStruct((NN, DD), jnp.float32),
            jax.ShapeDtypeStruct((2, DD), jnp.float32),
        ],
        scratch_shapes=[pltpu.VMEM((2, DD), jnp.float32)],
    )(r2a, r2b, g2a, g2b, dinv, b1)


def _d2_body(h_ref, stat_ref, gamma_ref, beta_ref, out_ref):
    mean = stat_ref[0:1] * (1.0 / NN)
    var = stat_ref[1:2] * (1.0 / NN) - mean * mean
    rstd = lax.rsqrt(var + 1e-5)
    out_ref[...] = (h_ref[...] - mean) * rstd * gamma_ref[...] + beta_ref[...]


def _d2(h2, stat, gamma, beta):
    return pl.pallas_call(
        _d2_body,
        grid=(GRID,),
        in_specs=[
            pl.BlockSpec((BLK, DD), lambda i: (i, 0)),
            pl.BlockSpec((2, DD), lambda i: (0, 0)),
            pl.BlockSpec((1, DD), lambda i: (0, 0)),
            pl.BlockSpec((1, DD), lambda i: (0, 0)),
        ],
        out_specs=pl.BlockSpec((BLK, DD), lambda i: (i, 0)),
        out_shape=jax.ShapeDtypeStruct((NN, DD), jnp.float32),
    )(h2, stat, gamma, beta)


def kernel(x, edge_index, W0, b0, W1, b1, gamma, beta):
    ei = edge_index.astype(jnp.int32)
    src, dst = ei[0], ei[1]
    ones = jnp.ones((CH, 16), jnp.float32)
    z1 = jnp.zeros((3128, 16), jnp.float32)
    zrows = jnp.zeros((184, DH), jnp.float32)

    deg2 = _deg_kernel(dst, ones, z1)
    ga, gb, dinv = _b1(x, W0, deg2)
    ra, rb = _agg_kernel(ga, gb, src, dst, zrows)
    g2a, g2b = _b2(ra, rb, ga, gb, dinv, W1, b0.reshape(1, DD))
    r2a, r2b = _agg_kernel(g2a, g2b, src, dst, zrows)
    h2, stat = _d1(r2a, r2b, g2a, g2b, dinv, b1.reshape(1, DD))
    out = _d2(h2, stat, gamma.reshape(1, DD), beta.reshape(1, DD))
    return (out[:NUSERS], out[NUSERS:])
